# Initial kernel scaffold; baseline (speedup 1.0000x reference)
import jax
import jax.numpy as jnp
from jax.experimental import pallas as pl


def kernel(predictions, labels, device):
    raise NotImplementedError("write your pallas kernel here")


# speedup vs baseline: 89.3592x; 89.3592x over previous
import jax
import jax.numpy as jnp
from jax import lax
from jax.experimental import pallas as pl
from jax.experimental.pallas import tpu as pltpu

_B = 256
_NH = 40
_NV = 40
_N = _NH * _NV
_THR = 0.025


def _loss_kernel(pred_ref, lab_ref, out_ref):
    p0 = pred_ref[:, 0, :]
    ep = jnp.exp(p0)
    e1p = jnp.exp(1.0 - p0)
    conf = ep / (ep + e1p)
    pos_log = -jnp.log(conf + 1e-8)
    neg_log = -jnp.log(1.0 - conf + 1e-8)

    P = jnp.sum(pos_log, axis=0, keepdims=True)
    Ng = jnp.sum(neg_log, axis=0, keepdims=True)
    T = jnp.sum(Ng)

    idx = lax.broadcasted_iota(jnp.int32, (1, _N), 1)
    gx = (idx // _NV).astype(jnp.float32) * (1.0 / _NH) + (0.5 / _NH)
    gy = (idx % _NV).astype(jnp.float32) * (1.0 / _NV) + (0.5 / _NV)

    lx = lab_ref[:, 0:1]
    ly = lab_ref[:, 1:2]
    dist = jnp.abs(gx - lx) + jnp.abs(gy - ly)
    pos = (dist <= _THR).astype(jnp.float32)

    num_pos = jnp.sum(pos, axis=1, keepdims=True)
    num_neg = jnp.float32(_N) - num_pos
    s_pos = jnp.sum(P * pos, axis=1, keepdims=True)
    s_negpos = jnp.sum(Ng * pos, axis=1, keepdims=True)

    loss = s_pos / num_pos + 3.0 * (T - s_negpos) / num_neg
    out_ref[0, 0] = jnp.sum(loss) * (1.0 / _B)


def kernel(predictions, labels, device):
    out = pl.pallas_call(
        _loss_kernel,
        grid=(),
        in_specs=[
            pl.BlockSpec((_B, 3, _N), lambda: (0, 0, 0)),
            pl.BlockSpec((_B, 2), lambda: (0, 0)),
        ],
        out_specs=pl.BlockSpec(memory_space=pltpu.SMEM),
        out_shape=jax.ShapeDtypeStruct((1, 1), jnp.float32),
    )(predictions, labels)
    return out[0, 0]


# speedup vs baseline: 91.8424x; 1.0278x over previous
import jax
import jax.numpy as jnp
from jax import lax
from jax.experimental import pallas as pl
from jax.experimental.pallas import tpu as pltpu

_B = 256
_NH = 40
_NV = 40
_N = _NH * _NV
_THR = 0.025


def _loss_kernel(pred_ref, lab_ref, out_ref):
    p0 = pred_ref[:, 0, :]
    t = jnp.exp(1.0 - 2.0 * p0)
    conf = 1.0 / (1.0 + t)
    pos_log = -jnp.log(conf + 1e-8)
    neg_log = -jnp.log(1.0 - conf + 1e-8)

    P = jnp.sum(pos_log, axis=0, keepdims=True)
    Ng = jnp.sum(neg_log, axis=0, keepdims=True)
    T = jnp.sum(Ng)

    idx = lax.broadcasted_iota(jnp.int32, (1, _N), 1)
    gx = (idx // _NV).astype(jnp.float32) * (1.0 / _NH) + (0.5 / _NH)
    gy = (idx % _NV).astype(jnp.float32) * (1.0 / _NV) + (0.5 / _NV)

    lx = lab_ref[:, 0:1]
    ly = lab_ref[:, 1:2]
    dist = jnp.abs(gx - lx) + jnp.abs(gy - ly)
    pos = (dist <= _THR).astype(jnp.float32)

    num_pos = jnp.sum(pos, axis=1, keepdims=True)
    num_neg = jnp.float32(_N) - num_pos
    s_pos = jnp.sum(P * pos, axis=1, keepdims=True)
    s_negpos = jnp.sum(Ng * pos, axis=1, keepdims=True)

    loss = s_pos / num_pos + 3.0 * (T - s_negpos) / num_neg
    out_ref[0, 0] = jnp.sum(loss) * (1.0 / _B)


def kernel(predictions, labels, device):
    out = pl.pallas_call(
        _loss_kernel,
        grid=(),
        in_specs=[
            pl.BlockSpec((_B, 3, _N), lambda: (0, 0, 0)),
            pl.BlockSpec((_B, 2), lambda: (0, 0)),
        ],
        out_specs=pl.BlockSpec(memory_space=pltpu.SMEM),
        out_shape=jax.ShapeDtypeStruct((1, 1), jnp.float32),
    )(predictions, labels)
    return out[0, 0]
